# Initial kernel scaffold; baseline (speedup 1.0000x reference)
import functools

import jax
import jax.numpy as jnp
from jax import lax
from jax.experimental import pallas as pl
from jax.experimental.pallas import tpu as pltpu
from jax.experimental.pallas import tpu_sc as plsc

N = 10000
D = 128
H = 256
OUT = 64

NPAD = 10240
BI = 512
NPB = NPAD // BI

CORES = 2
SUB = 16
K_CHUNK = 6
N_CHUNK = 27
RS = K_CHUNK * N_CHUNK
R_ROWS = RS * SUB
EP = R_ROWS * 128
DEG_ROWS = R_ROWS // (SUB * CORES)
ROWS_PER_SUB = NPAD // SUB

_mesh = plsc.VectorSubcoreMesh(core_axis_name="c", subcore_axis_name="s")


def _deg_body(dst_hbm, zeros16_hbm, ones16_hbm, degp_hbm,
              idx_v, ones_v, acc_sh, sem):
    c = lax.axis_index("c")
    s = lax.axis_index("s")
    pltpu.sync_copy(zeros16_hbm.at[pl.ds(s * ROWS_PER_SUB, ROWS_PER_SUB)],
                    acc_sh.at[pl.ds(s * ROWS_PER_SUB, ROWS_PER_SUB)])
    pltpu.sync_copy(ones16_hbm, ones_v)
    base = (c * SUB + s) * DEG_ROWS
    pltpu.sync_copy(dst_hbm.at[pl.ds(base, DEG_ROWS)], idx_v)
    plsc.subcore_barrier()

    @pl.loop(0, DEG_ROWS, step=9)
    def _(r0):
        descs = [
            pltpu.async_copy(ones_v, acc_sh.at[idx_v.at[r0 + j]], sem,
                             add=True)
            for j in range(9)
        ]
        for d in descs:
            d.wait()

    plsc.subcore_barrier()
    pltpu.sync_copy(
        acc_sh.at[pl.ds(s * ROWS_PER_SUB, ROWS_PER_SUB)],
        degp_hbm.at[pl.ds(c * NPAD + s * ROWS_PER_SUB, ROWS_PER_SUB)])


def _deg_call(dst2d, zeros16, ones16):
    return pl.kernel(
        _deg_body,
        out_type=jax.ShapeDtypeStruct((CORES * NPAD, 16), jnp.float32),
        mesh=_mesh,
        scratch_types=[
            pltpu.VMEM((DEG_ROWS, 128), jnp.int32),
            pltpu.VMEM((128, 16), jnp.float32),
            pltpu.VMEM_SHARED((NPAD, 16), jnp.float32),
            pltpu.SemaphoreType.DMA,
        ],
    )(dst2d, zeros16, ones16)


def _prop_body(g_hbm, srcs_hbm, dst_hbm, zeros_hbm, s_hbm,
               src_v, dst_v, msg_v, acc_sh, gsem, ssem):
    c = lax.axis_index("c")
    s = lax.axis_index("s")
    pltpu.sync_copy(zeros_hbm.at[pl.ds(s * ROWS_PER_SUB, ROWS_PER_SUB)],
                    acc_sh.at[pl.ds(s * ROWS_PER_SUB, ROWS_PER_SUB)])
    plsc.subcore_barrier()
    sub_base = s * RS

    @pl.loop(0, N_CHUNK)
    def _(ci):
        r0 = sub_base + ci * K_CHUNK
        pltpu.sync_copy(srcs_hbm.at[pl.ds(c * R_ROWS + r0, K_CHUNK)], src_v)
        pltpu.sync_copy(dst_hbm.at[pl.ds(r0, K_CHUNK)], dst_v)
        gd = [
            pltpu.async_copy(g_hbm.at[src_v.at[j]],
                             msg_v.at[pl.ds(j * 128, 128)], gsem)
            for j in range(K_CHUNK)
        ]
        for d in gd:
            d.wait()
        sd = [
            pltpu.async_copy(msg_v.at[pl.ds(j * 128, 128)],
                             acc_sh.at[dst_v.at[j]], ssem, add=True)
            for j in range(K_CHUNK)
        ]
        for d in sd:
            d.wait()

    plsc.subcore_barrier()
    pltpu.sync_copy(
        acc_sh.at[pl.ds(s * ROWS_PER_SUB, ROWS_PER_SUB)],
        s_hbm.at[pl.ds(c * NPAD + s * ROWS_PER_SUB, ROWS_PER_SUB)])


def _prop_call(g, srcs, dst2d, zeros):
    return pl.kernel(
        _prop_body,
        out_type=jax.ShapeDtypeStruct((CORES * NPAD, 128), jnp.float32),
        mesh=_mesh,
        scratch_types=[
            pltpu.VMEM((K_CHUNK, 128), jnp.int32),
            pltpu.VMEM((K_CHUNK, 128), jnp.int32),
            pltpu.VMEM((K_CHUNK * 128, 128), jnp.float32),
            pltpu.VMEM_SHARED((NPAD, 128), jnp.float32),
            pltpu.SemaphoreType.DMA,
            pltpu.SemaphoreType.DMA,
        ],
    )(g, srcs, dst2d, zeros)


def _dinv_body(degp_ref, dinv_ref):
    deg = degp_ref[0] + degp_ref[1]
    dinv_ref[...] = jnp.where(deg > 0.0, lax.rsqrt(deg), 0.0)


def _dinv_call(degp3d):
    return pl.pallas_call(
        _dinv_body,
        out_shape=jax.ShapeDtypeStruct((NPAD * 16 // 128, 128), jnp.float32),
    )(degp3d)


def _mm_body(s_ref, b_ref, dinv_ref, w_ref, g_ref, *, kh, prologue):
    k = pl.program_id(2)
    x_blk = s_ref[...]
    if prologue:
        x_blk = jnp.maximum(x_blk * dinv_ref[...] + b_ref[...], 0.0)

    @pl.when(k == 0)
    def _():
        g_ref[...] = jnp.zeros_like(g_ref)

    g_ref[...] += jnp.dot(x_blk, w_ref[...],
                          preferred_element_type=jnp.float32)

    @pl.when(k == kh - 1)
    def _():
        g_ref[...] *= dinv_ref[...]


def _mm_call(xin, b_prev, dinv, w, *, kh, prologue):
    body = functools.partial(_mm_body, kh=kh, prologue=prologue)
    return pl.pallas_call(
        body,
        grid=(CORES, NPB, kh),
        in_specs=[
            pl.BlockSpec((BI, 128), lambda c, i, k: (k * NPB + i, 0)),
            pl.BlockSpec((1, 128), lambda c, i, k: (0, k)),
            pl.BlockSpec((BI, 1), lambda c, i, k: (i, 0)),
            pl.BlockSpec((128, 128), lambda c, i, k: (k, c)),
        ],
        out_specs=pl.BlockSpec((BI, 128), lambda c, i, k: (c * NPB + i, 0)),
        out_shape=jax.ShapeDtypeStruct((CORES * NPAD, 128), jnp.float32),
    )(xin, b_prev, dinv, w)


def _jk_body(s1a, s1b, s2a, s2b, s3a, s3b, dinv_ref, b0_ref, b1_ref, b2_ref,
             w1_ref, w2_ref, w3a_ref, w3b_ref, blin_ref, pred_ref):
    dinv = dinv_ref[...]

    def halves(sa, sb, b_ref):
        xa = jnp.maximum(sa[...] * dinv + b_ref[:, 0:128], 0.0)
        xb = jnp.maximum(sb[...] * dinv + b_ref[:, 128:256], 0.0)
        return xa, xb

    x1a, x1b = halves(s1a, s1b, b0_ref)
    x2a, x2b = halves(s2a, s2b, b1_ref)
    x3a, x3b = halves(s3a, s3b, b2_ref)
    dot = functools.partial(jnp.dot, preferred_element_type=jnp.float32)
    acc = dot(x1a, w1_ref[0:128, :]) + dot(x1b, w1_ref[128:256, :])
    acc += dot(x2a, w2_ref[0:128, :]) + dot(x2b, w2_ref[128:256, :])
    acc += dot(x3a, w3a_ref[...]) + dot(x3b, w3b_ref[...])
    pred_ref[...] = acc + blin_ref[...]


def _jk_call(s1, s2, s3, dinv, b0p, b1p, b2p, w1l, w2l, w3la, w3lb, blinp):
    blk_a = pl.BlockSpec((BI, 128), lambda i: (i, 0))
    blk_b = pl.BlockSpec((BI, 128), lambda i: (NPB + i, 0))
    return pl.pallas_call(
        _jk_body,
        grid=(NPB,),
        in_specs=[
            blk_a, blk_b,
            blk_a, blk_b,
            blk_a, blk_b,
            pl.BlockSpec((BI, 1), lambda i: (i, 0)),
            pl.BlockSpec((1, 256), lambda i: (0, 0)),
            pl.BlockSpec((1, 256), lambda i: (0, 0)),
            pl.BlockSpec((1, 256), lambda i: (0, 0)),
            pl.BlockSpec((256, 128), lambda i: (0, 0)),
            pl.BlockSpec((256, 128), lambda i: (0, 0)),
            pl.BlockSpec((128, 128), lambda i: (0, 0)),
            pl.BlockSpec((128, 128), lambda i: (0, 0)),
            pl.BlockSpec((1, 128), lambda i: (0, 0)),
        ],
        out_specs=pl.BlockSpec((BI, 128), lambda i: (i, 0)),
        out_shape=jax.ShapeDtypeStruct((NPAD, 128), jnp.float32),
    )(s1, s1, s2, s2, s3, s3, dinv, b0p, b1p, b2p,
      w1l, w2l, w3la, w3lb, blinp)


@jax.jit
def _run(x, edge_index, W0, b0, W1, b1, W2, b2, Wlin, blin):
    ei = edge_index.astype(jnp.int32)
    e = ei.shape[1]
    loop_idx = jnp.arange(N, dtype=jnp.int32)
    pad_e = EP - (e + N)
    pad = jnp.full((pad_e,), NPAD - 1, jnp.int32)
    src = jnp.concatenate([ei[0], loop_idx, pad])
    dst = jnp.concatenate([ei[1], loop_idx, pad])
    srcs = jnp.concatenate([src, src + NPAD]).reshape(2 * R_ROWS, 128)
    dst2d = dst.reshape(R_ROWS, 128)

    zeros = jnp.zeros((NPAD, 128), jnp.float32)
    zeros16 = jnp.zeros((NPAD, 16), jnp.float32)
    ones16 = jnp.ones((128, 16), jnp.float32)
    x_p = jnp.zeros((NPAD, D), jnp.float32).at[:N].set(x)

    degp = _deg_call(dst2d, zeros16, ones16)
    dinvw = _dinv_call(degp.reshape(2, NPAD * 16 // 128, 128))
    dinv = dinvw.reshape(NPAD, 16)[:, 0:1]

    b0p = b0.reshape(1, H)
    b1p = b1.reshape(1, H)
    b2p = b2.reshape(1, H)
    zb = jnp.zeros((1, H), jnp.float32)

    g1 = _mm_call(x_p, zb, dinv, W0, kh=1, prologue=False)
    s1 = _prop_call(g1, srcs, dst2d, zeros)
    g2 = _mm_call(s1, b0p, dinv, W1, kh=2, prologue=True)
    s2 = _prop_call(g2, srcs, dst2d, zeros)
    g3 = _mm_call(s2, b1p, dinv, W2, kh=2, prologue=True)
    s3 = _prop_call(g3, srcs, dst2d, zeros)

    wp = jnp.pad(Wlin, ((0, 0), (0, 128 - OUT)))
    blinp = jnp.pad(blin, (0, 128 - OUT)).reshape(1, 128)
    pred = _jk_call(s1, s2, s3, dinv, b0p, b1p, b2p,
                    wp[0:256], wp[256:512], wp[512:640], wp[640:768], blinp)
    return pred[:N, :OUT]


def kernel(x, edge_index, W0, b0, W1, b1, W2, b2, Wlin, blin):
    return _run(x, edge_index, W0, b0, W1, b1, W2, b2, Wlin, blin)


# speedup vs baseline: 4.8526x; 4.8526x over previous
import functools

import jax
import jax.numpy as jnp
from jax import lax
from jax.experimental import pallas as pl
from jax.experimental.pallas import tpu as pltpu
from jax.experimental.pallas import tpu_sc as plsc

N = 10000
D = 128
H = 256
OUT = 64

NPAD = 10240
BI = 512
NPB = NPAD // BI

CORES = 2
SUB = 16
K_CHUNK = 8
N_CHUNK = 21
RS = K_CHUNK * N_CHUNK
R_ROWS = RS * SUB
EP = R_ROWS * 128
DEG_ROWS = 96
DEG_WORKERS = R_ROWS // DEG_ROWS
ROWS_PER_SUB = NPAD // SUB

@functools.cache
def _mesh():
    return plsc.VectorSubcoreMesh(core_axis_name="c", subcore_axis_name="s",
                                  num_cores=CORES, num_subcores=SUB)


def _deg_body(dst_hbm, zeros_hbm, ones_hbm, degp_hbm,
              idx_v, ones_v, acc_sh, sem):
    c = lax.axis_index("c")
    s = lax.axis_index("s")
    pltpu.sync_copy(zeros_hbm.at[pl.ds(s * ROWS_PER_SUB, ROWS_PER_SUB)],
                    acc_sh.at[pl.ds(s * ROWS_PER_SUB, ROWS_PER_SUB)])
    pltpu.sync_copy(ones_hbm, ones_v)
    w = c * SUB + s
    plsc.subcore_barrier()

    @pl.when(w < DEG_WORKERS)
    def _():
        base = w * DEG_ROWS

        @pl.loop(0, DEG_ROWS, step=8)
        def _(r0):
            pltpu.sync_copy(dst_hbm.at[pl.ds(base + r0, 8)], idx_v)
            descs = [
                pltpu.async_copy(ones_v, acc_sh.at[idx_v.at[j]], sem,
                                 add=True)
                for j in range(8)
            ]
            for d in descs:
                d.wait()

    plsc.subcore_barrier()
    pltpu.sync_copy(
        acc_sh.at[pl.ds(s * ROWS_PER_SUB, ROWS_PER_SUB)],
        degp_hbm.at[pl.ds(c * NPAD + s * ROWS_PER_SUB, ROWS_PER_SUB)])


def _deg_call(dst2d, zeros, ones):
    return pl.kernel(
        _deg_body,
        out_type=jax.ShapeDtypeStruct((CORES * NPAD, 128), jnp.float32),
        mesh=_mesh(),
        scratch_types=[
            pltpu.VMEM((8, 128), jnp.int32),
            pltpu.VMEM((128, 128), jnp.float32),
            pltpu.VMEM_SHARED((NPAD, 128), jnp.float32),
            pltpu.SemaphoreType.DMA,
        ],
    )(dst2d, zeros, ones)


def _prop_body(g_hbm, srcs_hbm, dst_hbm, zeros_hbm, s_hbm,
               src_v, dst_v, msg_v, acc_sh, gsem, ssem):
    c = lax.axis_index("c")
    s = lax.axis_index("s")
    pltpu.sync_copy(zeros_hbm.at[pl.ds(s * ROWS_PER_SUB, ROWS_PER_SUB)],
                    acc_sh.at[pl.ds(s * ROWS_PER_SUB, ROWS_PER_SUB)])
    plsc.subcore_barrier()
    sub_base = s * RS

    @pl.loop(0, N_CHUNK)
    def _(ci):
        r0 = sub_base + ci * K_CHUNK
        pltpu.sync_copy(srcs_hbm.at[pl.ds(c * R_ROWS + r0, K_CHUNK)], src_v)
        pltpu.sync_copy(dst_hbm.at[pl.ds(r0, K_CHUNK)], dst_v)
        for h in range(4):
            gd = [
                pltpu.async_copy(g_hbm.at[src_v.at[h * 2 + j]],
                                 msg_v.at[pl.ds(j * 128, 128)], gsem)
                for j in range(2)
            ]
            for d in gd:
                d.wait()
            sd = [
                pltpu.async_copy(msg_v.at[pl.ds(j * 128, 128)],
                                 acc_sh.at[dst_v.at[h * 2 + j]], ssem,
                                 add=True)
                for j in range(2)
            ]
            for d in sd:
                d.wait()

    plsc.subcore_barrier()
    pltpu.sync_copy(
        acc_sh.at[pl.ds(s * ROWS_PER_SUB, ROWS_PER_SUB)],
        s_hbm.at[pl.ds(c * NPAD + s * ROWS_PER_SUB, ROWS_PER_SUB)])


def _prop_call(g, srcs, dst2d, zeros):
    return pl.kernel(
        _prop_body,
        out_type=jax.ShapeDtypeStruct((CORES * NPAD, 128), jnp.float32),
        mesh=_mesh(),
        scratch_types=[
            pltpu.VMEM((K_CHUNK, 128), jnp.int32),
            pltpu.VMEM((K_CHUNK, 128), jnp.int32),
            pltpu.VMEM((256, 128), jnp.float32),
            pltpu.VMEM_SHARED((NPAD, 128), jnp.float32),
            pltpu.SemaphoreType.DMA,
            pltpu.SemaphoreType.DMA,
        ],
    )(g, srcs, dst2d, zeros)


def _dinv_body(degp_ref, dinv_ref):
    deg = degp_ref[0] + degp_ref[1]
    dinv_ref[...] = jnp.where(deg > 0.0, lax.rsqrt(deg), 0.0)


def _dinv_call(degp3d):
    return pl.pallas_call(
        _dinv_body,
        out_shape=jax.ShapeDtypeStruct((NPAD, 128), jnp.float32),
    )(degp3d)


def _mm_body(s_ref, b_ref, dinv_ref, w_ref, g_ref, *, kh, prologue):
    k = pl.program_id(2)
    x_blk = s_ref[...]
    if prologue:
        x_blk = jnp.maximum(x_blk * dinv_ref[...] + b_ref[...], 0.0)

    @pl.when(k == 0)
    def _():
        g_ref[...] = jnp.zeros_like(g_ref)

    g_ref[...] += jnp.dot(x_blk, w_ref[...],
                          preferred_element_type=jnp.float32)

    @pl.when(k == kh - 1)
    def _():
        g_ref[...] *= dinv_ref[...]


def _mm_call(xin, b_prev, dinv, w, *, kh, prologue):
    body = functools.partial(_mm_body, kh=kh, prologue=prologue)
    return pl.pallas_call(
        body,
        grid=(CORES, NPB, kh),
        in_specs=[
            pl.BlockSpec((BI, 128), lambda c, i, k: (k * NPB + i, 0)),
            pl.BlockSpec((1, 128), lambda c, i, k: (0, k)),
            pl.BlockSpec((BI, 1), lambda c, i, k: (i, 0)),
            pl.BlockSpec((128, 128), lambda c, i, k: (k, c)),
        ],
        out_specs=pl.BlockSpec((BI, 128), lambda c, i, k: (c * NPB + i, 0)),
        out_shape=jax.ShapeDtypeStruct((CORES * NPAD, 128), jnp.float32),
    )(xin, b_prev, dinv, w)


def _jk_body(s1a, s1b, s2a, s2b, s3a, s3b, dinv_ref, b0_ref, b1_ref, b2_ref,
             w1_ref, w2_ref, w3a_ref, w3b_ref, blin_ref, pred_ref):
    dinv = dinv_ref[...]

    def halves(sa, sb, b_ref):
        xa = jnp.maximum(sa[...] * dinv + b_ref[:, 0:128], 0.0)
        xb = jnp.maximum(sb[...] * dinv + b_ref[:, 128:256], 0.0)
        return xa, xb

    x1a, x1b = halves(s1a, s1b, b0_ref)
    x2a, x2b = halves(s2a, s2b, b1_ref)
    x3a, x3b = halves(s3a, s3b, b2_ref)
    dot = functools.partial(jnp.dot, preferred_element_type=jnp.float32)
    acc = dot(x1a, w1_ref[0:128, :]) + dot(x1b, w1_ref[128:256, :])
    acc += dot(x2a, w2_ref[0:128, :]) + dot(x2b, w2_ref[128:256, :])
    acc += dot(x3a, w3a_ref[...]) + dot(x3b, w3b_ref[...])
    pred_ref[...] = acc + blin_ref[...]


def _jk_call(s1, s2, s3, dinv, b0p, b1p, b2p, w1l, w2l, w3la, w3lb, blinp):
    blk_a = pl.BlockSpec((BI, 128), lambda i: (i, 0))
    blk_b = pl.BlockSpec((BI, 128), lambda i: (NPB + i, 0))
    return pl.pallas_call(
        _jk_body,
        grid=(NPB,),
        in_specs=[
            blk_a, blk_b,
            blk_a, blk_b,
            blk_a, blk_b,
            pl.BlockSpec((BI, 1), lambda i: (i, 0)),
            pl.BlockSpec((1, 256), lambda i: (0, 0)),
            pl.BlockSpec((1, 256), lambda i: (0, 0)),
            pl.BlockSpec((1, 256), lambda i: (0, 0)),
            pl.BlockSpec((256, 128), lambda i: (0, 0)),
            pl.BlockSpec((256, 128), lambda i: (0, 0)),
            pl.BlockSpec((128, 128), lambda i: (0, 0)),
            pl.BlockSpec((128, 128), lambda i: (0, 0)),
            pl.BlockSpec((1, 128), lambda i: (0, 0)),
        ],
        out_specs=pl.BlockSpec((BI, 128), lambda i: (i, 0)),
        out_shape=jax.ShapeDtypeStruct((NPAD, 128), jnp.float32),
    )(s1, s1, s2, s2, s3, s3, dinv, b0p, b1p, b2p,
      w1l, w2l, w3la, w3lb, blinp)


@jax.jit
def _run(x, edge_index, W0, b0, W1, b1, W2, b2, Wlin, blin):
    ei = edge_index.astype(jnp.int32)
    e = ei.shape[1]
    loop_idx = jnp.arange(N, dtype=jnp.int32)
    pad_e = EP - (e + N)
    pad = jnp.full((pad_e,), NPAD - 1, jnp.int32)
    src = jnp.concatenate([ei[0], loop_idx, pad])
    dst = jnp.concatenate([ei[1], loop_idx, pad])
    srcs = jnp.concatenate([src, src + NPAD]).reshape(2 * R_ROWS, 128)
    dst2d = dst.reshape(R_ROWS, 128)

    zeros = jnp.zeros((NPAD, 128), jnp.float32)
    ones = jnp.ones((128, 128), jnp.float32)
    x_p = jnp.zeros((NPAD, D), jnp.float32).at[:N].set(x)

    degp = _deg_call(dst2d, zeros, ones)
    dinvw = _dinv_call(degp.reshape(2, NPAD, 128))
    dinv = dinvw[:, 0:1]

    b0p = b0.reshape(1, H)
    b1p = b1.reshape(1, H)
    b2p = b2.reshape(1, H)
    zb = jnp.zeros((1, H), jnp.float32)

    g1 = _mm_call(x_p, zb, dinv, W0, kh=1, prologue=False)
    s1 = _prop_call(g1, srcs, dst2d, zeros)
    g2 = _mm_call(s1, b0p, dinv, W1, kh=2, prologue=True)
    s2 = _prop_call(g2, srcs, dst2d, zeros)
    g3 = _mm_call(s2, b1p, dinv, W2, kh=2, prologue=True)
    s3 = _prop_call(g3, srcs, dst2d, zeros)

    wp = jnp.pad(Wlin, ((0, 0), (0, 128 - OUT)))
    blinp = jnp.pad(blin, (0, 128 - OUT)).reshape(1, 128)
    pred = _jk_call(s1, s2, s3, dinv, b0p, b1p, b2p,
                    wp[0:256], wp[256:512], wp[512:640], wp[640:768], blinp)
    return pred[:N, :OUT]


def kernel(x, edge_index, W0, b0, W1, b1, W2, b2, Wlin, blin):
    return _run(x, edge_index, W0, b0, W1, b1, W2, b2, Wlin, blin)


# speedup vs baseline: 5.1931x; 1.0702x over previous
import functools

import jax
import jax.numpy as jnp
from jax import lax
from jax.experimental import pallas as pl
from jax.experimental.pallas import tpu as pltpu
from jax.experimental.pallas import tpu_sc as plsc

N = 10000
D = 128
H = 256
OUT = 64

NPAD = 10240
BI = 512
NPB = NPAD // BI

CORES = 2
SUB = 16
K_CHUNK = 8
N_CHUNK = 21
RS = K_CHUNK * N_CHUNK
R_ROWS = RS * SUB
EP = R_ROWS * 128
DEG_ROWS = 96
DEG_WORKERS = R_ROWS // DEG_ROWS
ROWS_PER_SUB = NPAD // SUB

@functools.cache
def _mesh():
    return plsc.VectorSubcoreMesh(core_axis_name="c", subcore_axis_name="s",
                                  num_cores=CORES, num_subcores=SUB)


def _deg_body(dst_hbm, zeros_hbm, ones_hbm, degp_hbm,
              idx_v, ones_v, acc_sh, sem):
    c = lax.axis_index("c")
    s = lax.axis_index("s")
    pltpu.sync_copy(zeros_hbm.at[pl.ds(s * ROWS_PER_SUB, ROWS_PER_SUB)],
                    acc_sh.at[pl.ds(s * ROWS_PER_SUB, ROWS_PER_SUB)])
    pltpu.sync_copy(ones_hbm, ones_v)
    w = c * SUB + s
    plsc.subcore_barrier()

    @pl.when(w < DEG_WORKERS)
    def _():
        base = w * DEG_ROWS

        @pl.loop(0, DEG_ROWS, step=8)
        def _(r0):
            pltpu.sync_copy(dst_hbm.at[pl.ds(base + r0, 8)], idx_v)
            descs = [
                pltpu.async_copy(ones_v, acc_sh.at[idx_v.at[j]], sem,
                                 add=True)
                for j in range(8)
            ]
            for d in descs:
                d.wait()

    plsc.subcore_barrier()
    pltpu.sync_copy(
        acc_sh.at[pl.ds(s * ROWS_PER_SUB, ROWS_PER_SUB)],
        degp_hbm.at[pl.ds(c * NPAD + s * ROWS_PER_SUB, ROWS_PER_SUB)])


def _deg_call(dst2d, zeros, ones):
    return pl.kernel(
        _deg_body,
        out_type=jax.ShapeDtypeStruct((CORES * NPAD, 128), jnp.float32),
        mesh=_mesh(),
        scratch_types=[
            pltpu.VMEM((8, 128), jnp.int32),
            pltpu.VMEM((128, 128), jnp.float32),
            pltpu.VMEM_SHARED((NPAD, 128), jnp.float32),
            pltpu.SemaphoreType.DMA,
        ],
    )(dst2d, zeros, ones)


def _prop_body(g_hbm, srcs_hbm, dst_hbm, zeros_hbm, s_hbm,
               src_v, dst_v, msg_v, acc_sh, gsem0, gsem1, ssem0, ssem1):
    c = lax.axis_index("c")
    s = lax.axis_index("s")
    pltpu.sync_copy(zeros_hbm.at[pl.ds(s * ROWS_PER_SUB, ROWS_PER_SUB)],
                    acc_sh.at[pl.ds(s * ROWS_PER_SUB, ROWS_PER_SUB)])
    plsc.subcore_barrier()
    sub_base = s * RS

    @pl.loop(0, N_CHUNK)
    def _(ci):
        r0 = sub_base + ci * K_CHUNK
        pltpu.sync_copy(srcs_hbm.at[pl.ds(c * R_ROWS + r0, K_CHUNK)], src_v)
        pltpu.sync_copy(dst_hbm.at[pl.ds(r0, K_CHUNK)], dst_v)
        msgs = (msg_v.at[pl.ds(0, 128)], msg_v.at[pl.ds(128, 128)])
        gsems = (gsem0, gsem1)
        ssems = (ssem0, ssem1)
        gd = [None] * K_CHUNK
        for j in range(2):
            gd[j] = pltpu.async_copy(g_hbm.at[src_v.at[j]], msgs[j],
                                     gsems[j])
        for j in range(K_CHUNK):
            sl = j % 2
            gd[j].wait()
            sd = pltpu.async_copy(msgs[sl], acc_sh.at[dst_v.at[j]],
                                  ssems[sl], add=True)
            sd.wait()
            if j + 2 < K_CHUNK:
                gd[j + 2] = pltpu.async_copy(g_hbm.at[src_v.at[j + 2]],
                                             msgs[sl], gsems[sl])

    plsc.subcore_barrier()
    pltpu.sync_copy(
        acc_sh.at[pl.ds(s * ROWS_PER_SUB, ROWS_PER_SUB)],
        s_hbm.at[pl.ds(c * NPAD + s * ROWS_PER_SUB, ROWS_PER_SUB)])


def _prop_call(g, srcs, dst2d, zeros):
    return pl.kernel(
        _prop_body,
        out_type=jax.ShapeDtypeStruct((CORES * NPAD, 128), jnp.float32),
        mesh=_mesh(),
        scratch_types=[
            pltpu.VMEM((K_CHUNK, 128), jnp.int32),
            pltpu.VMEM((K_CHUNK, 128), jnp.int32),
            pltpu.VMEM((256, 128), jnp.float32),
            pltpu.VMEM_SHARED((NPAD, 128), jnp.float32),
            pltpu.SemaphoreType.DMA,
            pltpu.SemaphoreType.DMA,
            pltpu.SemaphoreType.DMA,
            pltpu.SemaphoreType.DMA,
        ],
    )(g, srcs, dst2d, zeros)


def _dinv_body(degp_ref, dinv_ref):
    deg = degp_ref[0] + degp_ref[1]
    dinv_ref[...] = jnp.where(deg > 0.0, lax.rsqrt(deg), 0.0)


def _dinv_call(degp3d):
    return pl.pallas_call(
        _dinv_body,
        out_shape=jax.ShapeDtypeStruct((NPAD, 128), jnp.float32),
    )(degp3d)


def _mm_body(s_ref, b_ref, dinv_ref, w_ref, g_ref, *, kh, prologue):
    k = pl.program_id(2)
    x_blk = s_ref[...]
    if prologue:
        x_blk = jnp.maximum(x_blk * dinv_ref[...] + b_ref[...], 0.0)

    @pl.when(k == 0)
    def _():
        g_ref[...] = jnp.zeros_like(g_ref)

    g_ref[...] += jnp.dot(x_blk, w_ref[...],
                          preferred_element_type=jnp.float32)

    @pl.when(k == kh - 1)
    def _():
        g_ref[...] *= dinv_ref[...]


def _mm_call(xin, b_prev, dinv, w, *, kh, prologue):
    body = functools.partial(_mm_body, kh=kh, prologue=prologue)
    return pl.pallas_call(
        body,
        grid=(CORES, NPB, kh),
        in_specs=[
            pl.BlockSpec((BI, 128), lambda c, i, k: (k * NPB + i, 0)),
            pl.BlockSpec((1, 128), lambda c, i, k: (0, k)),
            pl.BlockSpec((BI, 1), lambda c, i, k: (i, 0)),
            pl.BlockSpec((128, 128), lambda c, i, k: (k, c)),
        ],
        out_specs=pl.BlockSpec((BI, 128), lambda c, i, k: (c * NPB + i, 0)),
        out_shape=jax.ShapeDtypeStruct((CORES * NPAD, 128), jnp.float32),
    )(xin, b_prev, dinv, w)


def _jk_body(s1a, s1b, s2a, s2b, s3a, s3b, dinv_ref, b0_ref, b1_ref, b2_ref,
             w1_ref, w2_ref, w3a_ref, w3b_ref, blin_ref, pred_ref):
    dinv = dinv_ref[...]

    def halves(sa, sb, b_ref):
        xa = jnp.maximum(sa[...] * dinv + b_ref[:, 0:128], 0.0)
        xb = jnp.maximum(sb[...] * dinv + b_ref[:, 128:256], 0.0)
        return xa, xb

    x1a, x1b = halves(s1a, s1b, b0_ref)
    x2a, x2b = halves(s2a, s2b, b1_ref)
    x3a, x3b = halves(s3a, s3b, b2_ref)
    dot = functools.partial(jnp.dot, preferred_element_type=jnp.float32)
    acc = dot(x1a, w1_ref[0:128, :]) + dot(x1b, w1_ref[128:256, :])
    acc += dot(x2a, w2_ref[0:128, :]) + dot(x2b, w2_ref[128:256, :])
    acc += dot(x3a, w3a_ref[...]) + dot(x3b, w3b_ref[...])
    pred_ref[...] = acc + blin_ref[...]


def _jk_call(s1, s2, s3, dinv, b0p, b1p, b2p, w1l, w2l, w3la, w3lb, blinp):
    blk_a = pl.BlockSpec((BI, 128), lambda i: (i, 0))
    blk_b = pl.BlockSpec((BI, 128), lambda i: (NPB + i, 0))
    return pl.pallas_call(
        _jk_body,
        grid=(NPB,),
        in_specs=[
            blk_a, blk_b,
            blk_a, blk_b,
            blk_a, blk_b,
            pl.BlockSpec((BI, 1), lambda i: (i, 0)),
            pl.BlockSpec((1, 256), lambda i: (0, 0)),
            pl.BlockSpec((1, 256), lambda i: (0, 0)),
            pl.BlockSpec((1, 256), lambda i: (0, 0)),
            pl.BlockSpec((256, 128), lambda i: (0, 0)),
            pl.BlockSpec((256, 128), lambda i: (0, 0)),
            pl.BlockSpec((128, 128), lambda i: (0, 0)),
            pl.BlockSpec((128, 128), lambda i: (0, 0)),
            pl.BlockSpec((1, 128), lambda i: (0, 0)),
        ],
        out_specs=pl.BlockSpec((BI, 128), lambda i: (i, 0)),
        out_shape=jax.ShapeDtypeStruct((NPAD, 128), jnp.float32),
    )(s1, s1, s2, s2, s3, s3, dinv, b0p, b1p, b2p,
      w1l, w2l, w3la, w3lb, blinp)


@jax.jit
def _run(x, edge_index, W0, b0, W1, b1, W2, b2, Wlin, blin):
    ei = edge_index.astype(jnp.int32)
    e = ei.shape[1]
    loop_idx = jnp.arange(N, dtype=jnp.int32)
    pad_e = EP - (e + N)
    pad = jnp.full((pad_e,), NPAD - 1, jnp.int32)
    src = jnp.concatenate([ei[0], loop_idx, pad])
    dst = jnp.concatenate([ei[1], loop_idx, pad])
    srcs = jnp.concatenate([src, src + NPAD]).reshape(2 * R_ROWS, 128)
    dst2d = dst.reshape(R_ROWS, 128)

    zeros = jnp.zeros((NPAD, 128), jnp.float32)
    ones = jnp.ones((128, 128), jnp.float32)
    x_p = jnp.zeros((NPAD, D), jnp.float32).at[:N].set(x)

    degp = _deg_call(dst2d, zeros, ones)
    dinvw = _dinv_call(degp.reshape(2, NPAD, 128))
    dinv = dinvw[:, 0:1]

    b0p = b0.reshape(1, H)
    b1p = b1.reshape(1, H)
    b2p = b2.reshape(1, H)
    zb = jnp.zeros((1, H), jnp.float32)

    g1 = _mm_call(x_p, zb, dinv, W0, kh=1, prologue=False)
    s1 = _prop_call(g1, srcs, dst2d, zeros)
    g2 = _mm_call(s1, b0p, dinv, W1, kh=2, prologue=True)
    s2 = _prop_call(g2, srcs, dst2d, zeros)
    g3 = _mm_call(s2, b1p, dinv, W2, kh=2, prologue=True)
    s3 = _prop_call(g3, srcs, dst2d, zeros)

    wp = jnp.pad(Wlin, ((0, 0), (0, 128 - OUT)))
    blinp = jnp.pad(blin, (0, 128 - OUT)).reshape(1, 128)
    pred = _jk_call(s1, s2, s3, dinv, b0p, b1p, b2p,
                    wp[0:256], wp[256:512], wp[512:640], wp[640:768], blinp)
    return pred[:N, :OUT]


def kernel(x, edge_index, W0, b0, W1, b1, W2, b2, Wlin, blin):
    return _run(x, edge_index, W0, b0, W1, b1, W2, b2, Wlin, blin)


# speedup vs baseline: 7.3559x; 1.4165x over previous
import functools

import jax
import jax.numpy as jnp
from jax import lax
from jax.experimental import pallas as pl
from jax.experimental.pallas import tpu as pltpu
from jax.experimental.pallas import tpu_sc as plsc

N = 10000
D = 128
H = 256
OUT = 64

NPAD = 10240
BI = 512
NPB = NPAD // BI

CORES = 2
SUB = 16
K_CHUNK = 8
N_CHUNK = 20
RS = K_CHUNK * N_CHUNK
R_ROWS = RS * SUB
EP = R_ROWS * 128
DEG_ROWS = R_ROWS // (SUB * CORES)
ROWS_PER_SUB = NPAD // SUB

@functools.cache
def _mesh():
    return plsc.VectorSubcoreMesh(core_axis_name="c", subcore_axis_name="s",
                                  num_cores=CORES, num_subcores=SUB)


def _deg_body(dst_hbm, zeros_hbm, ones_hbm, degp_hbm,
              idx_v, ones_v, acc_sh, sem):
    c = lax.axis_index("c")
    s = lax.axis_index("s")
    pltpu.sync_copy(zeros_hbm.at[pl.ds(s * ROWS_PER_SUB, ROWS_PER_SUB)],
                    acc_sh.at[pl.ds(s * ROWS_PER_SUB, ROWS_PER_SUB)])
    pltpu.sync_copy(ones_hbm, ones_v)
    base = (c * SUB + s) * DEG_ROWS
    plsc.subcore_barrier()

    @pl.loop(0, DEG_ROWS, step=8)
    def _(r0):
        pltpu.sync_copy(dst_hbm.at[pl.ds(base + r0, 8)], idx_v)
        descs = [
            pltpu.async_copy(ones_v, acc_sh.at[idx_v.at[j]], sem, add=True)
            for j in range(8)
        ]
        for d in descs:
            d.wait()

    plsc.subcore_barrier()
    pltpu.sync_copy(
        acc_sh.at[pl.ds(s * ROWS_PER_SUB, ROWS_PER_SUB)],
        degp_hbm.at[pl.ds(c * NPAD + s * ROWS_PER_SUB, ROWS_PER_SUB)])


def _deg_call(dst2d, zeros, ones):
    return pl.kernel(
        _deg_body,
        out_type=jax.ShapeDtypeStruct((CORES * NPAD, 128), jnp.float32),
        mesh=_mesh(),
        scratch_types=[
            pltpu.VMEM((8, 128), jnp.int32),
            pltpu.VMEM((128, 128), jnp.float32),
            pltpu.VMEM_SHARED((NPAD, 128), jnp.float32),
            pltpu.SemaphoreType.DMA,
        ],
    )(dst2d, zeros, ones)


def _prop_body(g_hbm, srcs_hbm, dst_hbm, zeros_hbm, s_hbm,
               src_a, dst_a, src_b, dst_b, msg_v, acc_sh,
               gsem0, gsem1, ssem0, ssem1, isem_a, isem_b):
    c = lax.axis_index("c")
    s = lax.axis_index("s")
    pltpu.sync_copy(zeros_hbm.at[pl.ds(s * ROWS_PER_SUB, ROWS_PER_SUB)],
                    acc_sh.at[pl.ds(s * ROWS_PER_SUB, ROWS_PER_SUB)])
    plsc.subcore_barrier()
    sub_base = s * RS
    sbase = c * R_ROWS + sub_base
    msgs = (msg_v.at[pl.ds(0, 128)], msg_v.at[pl.ds(128, 128)])
    gsems = (gsem0, gsem1)
    ssems = (ssem0, ssem1)

    def fire_idx(ci, sbuf, dbuf, sem):
        pltpu.async_copy(srcs_hbm.at[pl.ds(sbase + ci * K_CHUNK, K_CHUNK)],
                         sbuf, sem)
        pltpu.async_copy(dst_hbm.at[pl.ds(sub_base + ci * K_CHUNK, K_CHUNK)],
                         dbuf, sem)

    def wait_idx(sbuf, dbuf, sem):
        pltpu.make_async_copy(srcs_hbm.at[pl.ds(0, K_CHUNK)], sbuf,
                              sem).wait()
        pltpu.make_async_copy(dst_hbm.at[pl.ds(0, K_CHUNK)], dbuf,
                              sem).wait()

    def process(sbuf, dbuf):
        gd = [None] * K_CHUNK
        for j in range(2):
            gd[j] = pltpu.async_copy(g_hbm.at[sbuf.at[j]], msgs[j],
                                     gsems[j])
        for j in range(K_CHUNK):
            sl = j % 2
            gd[j].wait()
            sd = pltpu.async_copy(msgs[sl], acc_sh.at[dbuf.at[j]],
                                  ssems[sl], add=True)
            sd.wait()
            if j + 2 < K_CHUNK:
                gd[j + 2] = pltpu.async_copy(g_hbm.at[sbuf.at[j + 2]],
                                             msgs[sl], gsems[sl])

    fire_idx(0, src_a, dst_a, isem_a)

    @pl.loop(0, N_CHUNK // 2)
    def _(t):
        c0 = 2 * t
        fire_idx(c0 + 1, src_b, dst_b, isem_b)
        wait_idx(src_a, dst_a, isem_a)
        process(src_a, dst_a)

        @pl.when(t < N_CHUNK // 2 - 1)
        def _():
            fire_idx(c0 + 2, src_a, dst_a, isem_a)

        wait_idx(src_b, dst_b, isem_b)
        process(src_b, dst_b)

    plsc.subcore_barrier()
    pltpu.sync_copy(
        acc_sh.at[pl.ds(s * ROWS_PER_SUB, ROWS_PER_SUB)],
        s_hbm.at[pl.ds(c * NPAD + s * ROWS_PER_SUB, ROWS_PER_SUB)])


def _prop_call(g, srcs, dst2d, zeros):
    return pl.kernel(
        _prop_body,
        out_type=jax.ShapeDtypeStruct((CORES * NPAD, 128), jnp.float32),
        mesh=_mesh(),
        scratch_types=[
            pltpu.VMEM((K_CHUNK, 128), jnp.int32),
            pltpu.VMEM((K_CHUNK, 128), jnp.int32),
            pltpu.VMEM((K_CHUNK, 128), jnp.int32),
            pltpu.VMEM((K_CHUNK, 128), jnp.int32),
            pltpu.VMEM((256, 128), jnp.float32),
            pltpu.VMEM_SHARED((NPAD, 128), jnp.float32),
            pltpu.SemaphoreType.DMA,
            pltpu.SemaphoreType.DMA,
            pltpu.SemaphoreType.DMA,
            pltpu.SemaphoreType.DMA,
            pltpu.SemaphoreType.DMA,
            pltpu.SemaphoreType.DMA,
        ],
    )(g, srcs, dst2d, zeros)


def _dinv_body(degp_ref, dinv_ref):
    deg = degp_ref[0] + degp_ref[1] + 1.0
    dinv_ref[...] = jnp.where(deg > 0.0, lax.rsqrt(deg), 0.0)


def _dinv_call(degp3d):
    return pl.pallas_call(
        _dinv_body,
        out_shape=jax.ShapeDtypeStruct((NPAD, 128), jnp.float32),
    )(degp3d)


def _mm_body(s_ref, gp_ref, b_ref, dinv_ref, w_ref, g_ref, *, kh, prologue):
    k = pl.program_id(2)
    x_blk = s_ref[...]
    if prologue:
        x_blk = jnp.maximum((x_blk + gp_ref[...]) * dinv_ref[...]
                            + b_ref[...], 0.0)

    @pl.when(k == 0)
    def _():
        g_ref[...] = jnp.zeros_like(g_ref)

    g_ref[...] += jnp.dot(x_blk, w_ref[...],
                          preferred_element_type=jnp.float32)

    @pl.when(k == kh - 1)
    def _():
        g_ref[...] *= dinv_ref[...]


def _mm_call(xin, g_prev, b_prev, dinv, w, *, kh, prologue):
    body = functools.partial(_mm_body, kh=kh, prologue=prologue)
    stk = pl.BlockSpec((BI, 128), lambda c, i, k: (k * NPB + i, 0))
    return pl.pallas_call(
        body,
        grid=(CORES, NPB, kh),
        in_specs=[
            stk,
            stk,
            pl.BlockSpec((1, 128), lambda c, i, k: (0, k)),
            pl.BlockSpec((BI, 1), lambda c, i, k: (i, 0)),
            pl.BlockSpec((128, 128), lambda c, i, k: (k, c)),
        ],
        out_specs=pl.BlockSpec((BI, 128), lambda c, i, k: (c * NPB + i, 0)),
        out_shape=jax.ShapeDtypeStruct((CORES * NPAD, 128), jnp.float32),
    )(xin, g_prev, b_prev, dinv, w)


def _jk_body(s1a, s1b, s2a, s2b, s3a, s3b, g1a, g1b, g2a, g2b, g3a, g3b,
             dinv_ref, b0_ref, b1_ref, b2_ref,
             w1_ref, w2_ref, w3a_ref, w3b_ref, blin_ref, pred_ref):
    dinv = dinv_ref[...]

    def halves(sa, sb, ga, gb, b_ref):
        xa = jnp.maximum((sa[...] + ga[...]) * dinv + b_ref[:, 0:128], 0.0)
        xb = jnp.maximum((sb[...] + gb[...]) * dinv + b_ref[:, 128:256], 0.0)
        return xa, xb

    x1a, x1b = halves(s1a, s1b, g1a, g1b, b0_ref)
    x2a, x2b = halves(s2a, s2b, g2a, g2b, b1_ref)
    x3a, x3b = halves(s3a, s3b, g3a, g3b, b2_ref)
    dot = functools.partial(jnp.dot, preferred_element_type=jnp.float32)
    acc = dot(x1a, w1_ref[0:128, :]) + dot(x1b, w1_ref[128:256, :])
    acc += dot(x2a, w2_ref[0:128, :]) + dot(x2b, w2_ref[128:256, :])
    acc += dot(x3a, w3a_ref[...]) + dot(x3b, w3b_ref[...])
    pred_ref[...] = acc + blin_ref[...]


def _jk_call(s1, s2, s3, g1, g2, g3, dinv, b0p, b1p, b2p,
             w1l, w2l, w3la, w3lb, blinp):
    blk_a = pl.BlockSpec((BI, 128), lambda i: (i, 0))
    blk_b = pl.BlockSpec((BI, 128), lambda i: (NPB + i, 0))
    return pl.pallas_call(
        _jk_body,
        grid=(NPB,),
        in_specs=[
            blk_a, blk_b,
            blk_a, blk_b,
            blk_a, blk_b,
            blk_a, blk_b,
            blk_a, blk_b,
            blk_a, blk_b,
            pl.BlockSpec((BI, 1), lambda i: (i, 0)),
            pl.BlockSpec((1, 256), lambda i: (0, 0)),
            pl.BlockSpec((1, 256), lambda i: (0, 0)),
            pl.BlockSpec((1, 256), lambda i: (0, 0)),
            pl.BlockSpec((256, 128), lambda i: (0, 0)),
            pl.BlockSpec((256, 128), lambda i: (0, 0)),
            pl.BlockSpec((128, 128), lambda i: (0, 0)),
            pl.BlockSpec((128, 128), lambda i: (0, 0)),
            pl.BlockSpec((1, 128), lambda i: (0, 0)),
        ],
        out_specs=pl.BlockSpec((BI, 128), lambda i: (i, 0)),
        out_shape=jax.ShapeDtypeStruct((NPAD, 128), jnp.float32),
    )(s1, s1, s2, s2, s3, s3, g1, g1, g2, g2, g3, g3, dinv, b0p, b1p, b2p,
      w1l, w2l, w3la, w3lb, blinp)


@jax.jit
def _run(x, edge_index, W0, b0, W1, b1, W2, b2, Wlin, blin):
    ei = edge_index.astype(jnp.int32)
    e = ei.shape[1]
    pad_e = EP - e
    pad = jnp.full((pad_e,), NPAD - 1, jnp.int32)
    src = jnp.concatenate([ei[0], pad])
    dst = jnp.concatenate([ei[1], pad])
    srcs = jnp.concatenate([src, src + NPAD]).reshape(2 * R_ROWS, 128)
    dst2d = dst.reshape(R_ROWS, 128)

    zeros = jnp.zeros((NPAD, 128), jnp.float32)
    ones = jnp.ones((128, 128), jnp.float32)
    x_p = jnp.zeros((NPAD, D), jnp.float32).at[:N].set(x)

    degp = _deg_call(dst2d, zeros, ones)
    dinvw = _dinv_call(degp.reshape(2, NPAD, 128))
    dinv = dinvw[:, 0:1]

    b0p = b0.reshape(1, H)
    b1p = b1.reshape(1, H)
    b2p = b2.reshape(1, H)
    zb = jnp.zeros((1, H), jnp.float32)

    g1 = _mm_call(x_p, x_p, zb, dinv, W0, kh=1, prologue=False)
    s1 = _prop_call(g1, srcs, dst2d, zeros)
    g2 = _mm_call(s1, g1, b0p, dinv, W1, kh=2, prologue=True)
    s2 = _prop_call(g2, srcs, dst2d, zeros)
    g3 = _mm_call(s2, g2, b1p, dinv, W2, kh=2, prologue=True)
    s3 = _prop_call(g3, srcs, dst2d, zeros)

    wp = jnp.pad(Wlin, ((0, 0), (0, 128 - OUT)))
    blinp = jnp.pad(blin, (0, 128 - OUT)).reshape(1, 128)
    pred = _jk_call(s1, s2, s3, g1, g2, g3, dinv, b0p, b1p, b2p,
                    wp[0:256], wp[256:512], wp[512:640], wp[640:768], blinp)
    return pred[:N, :OUT]


def kernel(x, edge_index, W0, b0, W1, b1, W2, b2, Wlin, blin):
    return _run(x, edge_index, W0, b0, W1, b1, W2, b2, Wlin, blin)
